# baseline (speedup 1.0000x reference)
import jax
import jax.numpy as jnp
from jax.experimental import pallas as pl
from jax.experimental.pallas import tpu as pltpu

_B, _N, _D = 4, 4096, 256
_BN = 256


def _proj_body(p_ref, x_ref, o_ref):
    p = p_ref[...]
    for b in range(_B):
        o_ref[b] = jnp.dot(p, x_ref[b], preferred_element_type=jnp.float32)


def kernel(x, projection_matrix):
    grid = (_N // _BN,)
    return pl.pallas_call(
        _proj_body,
        grid=grid,
        in_specs=[
            pl.BlockSpec((_BN, _N), lambda i: (i, 0)),
            pl.BlockSpec((_B, _N, _D), lambda i: (0, 0, 0)),
        ],
        out_specs=pl.BlockSpec((_B, _BN, _D), lambda i: (0, i, 0)),
        out_shape=jax.ShapeDtypeStruct((_B, _N, _D), jnp.float32),
        compiler_params=pltpu.CompilerParams(
            dimension_semantics=("arbitrary",),
        ),
    )(projection_matrix, x)


# speedup vs baseline: 1.0190x; 1.0190x over previous
import jax
import jax.numpy as jnp
from jax.experimental import pallas as pl
from jax.experimental.pallas import tpu as pltpu

_B, _N, _D = 4, 4096, 256
_BN = 1024


def _proj_body(p_ref, x_ref, o_ref):
    p = p_ref[...]
    for b in range(_B):
        o_ref[b] = jnp.dot(p, x_ref[b], preferred_element_type=jnp.float32)


def kernel(x, projection_matrix):
    grid = (_N // _BN,)
    return pl.pallas_call(
        _proj_body,
        grid=grid,
        in_specs=[
            pl.BlockSpec((_BN, _N), lambda i: (i, 0)),
            pl.BlockSpec((_B, _N, _D), lambda i: (0, 0, 0)),
        ],
        out_specs=pl.BlockSpec((_B, _BN, _D), lambda i: (0, i, 0)),
        out_shape=jax.ShapeDtypeStruct((_B, _N, _D), jnp.float32),
        compiler_params=pltpu.CompilerParams(
            dimension_semantics=("arbitrary",),
        ),
    )(projection_matrix, x)


# speedup vs baseline: 1.0608x; 1.0410x over previous
import jax
import jax.numpy as jnp
from jax.experimental import pallas as pl
from jax.experimental.pallas import tpu as pltpu

_B, _N, _D = 4, 4096, 256
_BN = 512


def _proj_body(p_ref, x_ref, o_ref):
    p = p_ref[...]
    for b in range(_B):
        o_ref[b] = jnp.dot(p, x_ref[b], preferred_element_type=jnp.float32)


def kernel(x, projection_matrix):
    grid = (_N // _BN,)
    return pl.pallas_call(
        _proj_body,
        grid=grid,
        in_specs=[
            pl.BlockSpec((_BN, _N), lambda i: (i, 0)),
            pl.BlockSpec((_B, _N, _D), lambda i: (0, 0, 0)),
        ],
        out_specs=pl.BlockSpec((_B, _BN, _D), lambda i: (0, i, 0)),
        out_shape=jax.ShapeDtypeStruct((_B, _N, _D), jnp.float32),
        compiler_params=pltpu.CompilerParams(
            dimension_semantics=("parallel",),
        ),
    )(projection_matrix, x)


# speedup vs baseline: 1.0608x; 1.0000x over previous
import jax
import jax.numpy as jnp
from jax.experimental import pallas as pl
from jax.experimental.pallas import tpu as pltpu

_BN = 512


def _make_body(batch):
    def _proj_body(p_ref, x_ref, o_ref):
        p = p_ref[...]
        for b in range(batch):
            o_ref[b] = jnp.dot(p, x_ref[b], preferred_element_type=jnp.float32)

    return _proj_body


def kernel(x, projection_matrix):
    B, N, D = x.shape
    grid = (N // _BN,)
    return pl.pallas_call(
        _make_body(B),
        grid=grid,
        in_specs=[
            pl.BlockSpec((_BN, N), lambda i: (i, 0)),
            pl.BlockSpec((B, N, D), lambda i: (0, 0, 0)),
        ],
        out_specs=pl.BlockSpec((B, _BN, D), lambda i: (0, i, 0)),
        out_shape=jax.ShapeDtypeStruct((B, N, D), jnp.float32),
        compiler_params=pltpu.CompilerParams(
            dimension_semantics=("parallel",),
        ),
    )(projection_matrix, x)
